# Initial kernel scaffold; baseline (speedup 1.0000x reference)
import jax
import jax.numpy as jnp
from jax.experimental import pallas as pl


def kernel(x, edge_index, W1, a1_src, a1_dst, b1, W2, a2_src, a2_dst, b2):
    raise NotImplementedError("write your pallas kernel here")


# speedup vs baseline: 31.4129x; 31.4129x over previous
import functools

import jax
import jax.numpy as jnp
from jax import lax
from jax.experimental import pallas as pl
from jax.experimental.pallas import tpu as pltpu
from jax.experimental.pallas import tpu_sc as plsc

N = 10000
NP = 10240
D = 128
E = 320000
K = 128
NB = 79
NW = 32
EW2 = NB * K
E2 = NW * EW2


def _sc_attention(src3, dst3, asrc, adst, h):
    mesh = plsc.VectorSubcoreMesh(core_axis_name="c", subcore_axis_name="s")

    @functools.partial(
        pl.kernel,
        mesh=mesh,
        out_type=[
            jax.ShapeDtypeStruct((2, 16, 640, D), jnp.float32),
            jax.ShapeDtypeStruct((2 * 16 * 640,), jnp.float32),
        ],
        scratch_types=[
            pltpu.VMEM((NB, K), jnp.int32),
            pltpu.VMEM((NB, K), jnp.int32),
            pltpu.VMEM((K,), jnp.float32),
            pltpu.VMEM((K,), jnp.float32),
            pltpu.VMEM((K,), jnp.float32),
            pltpu.VMEM((K, D), jnp.float32),
            pltpu.VMEM((K,), jnp.float32),
            pltpu.VMEM_SHARED((NP, D), jnp.float32),
            pltpu.VMEM_SHARED((NP,), jnp.float32),
            pltpu.SemaphoreType.DMA,
            pltpu.SemaphoreType.DMA,
            pltpu.SemaphoreType.DMA,
        ],
        compiler_params=pltpu.CompilerParams(needs_layout_passes=False),
    )
    def sc_kernel(src_h, dst_h, asrc_h, adst_h, h_h, outp_h, denp_h,
                  src_v, dst_v, asb, adb, exb, rows, db,
                  out_sp, den_sp, sem_r, sem_a, sem_d):
        c = lax.axis_index("c")
        s = lax.axis_index("s")
        wid = s * 2 + c

        zvec = jnp.zeros((16,), jnp.float32)

        def zrow(r, carry):
            for cc in range(8):
                rows[r, pl.ds(cc * 16, 16)] = zvec
            return carry

        lax.fori_loop(0, K, zrow, 0)
        for i in range(8):
            db[pl.ds(i * 16, 16)] = zvec

        zoff = s * 640
        for k in range(5):
            pltpu.sync_copy(rows, out_sp.at[pl.ds(zoff + k * 128, 128)])
            pltpu.sync_copy(db, den_sp.at[pl.ds(zoff + k * 128, 128)])

        pltpu.sync_copy(src_h.at[wid], src_v)
        pltpu.sync_copy(dst_h.at[wid], dst_v)

        plsc.subcore_barrier()

        def batch(j, carry):
            gr = pltpu.async_copy(h_h.at[src_v.at[j]], rows, sem_r)
            ga = pltpu.async_copy(asrc_h.at[src_v.at[j]], asb, sem_a)
            gd = pltpu.async_copy(adst_h.at[dst_v.at[j]], adb, sem_d)
            ga.wait()
            gd.wait()
            for t in range(8):
                e = (asb[pl.ds(t * 16, 16)] + adb[pl.ds(t * 16, 16)])
                e = jnp.maximum(e, e * 0.2)
                exb[pl.ds(t * 16, 16)] = jnp.exp(e)
            gr.wait()

            def scale(r, inner):
                ev = plsc.load_gather(exb, [jnp.full((16,), r, jnp.int32)])
                for cc in range(8):
                    rows[r, pl.ds(cc * 16, 16)] = (
                        rows[r, pl.ds(cc * 16, 16)] * ev)
                return inner

            lax.fori_loop(0, K, scale, 0, unroll=2)
            pltpu.sync_copy(rows, out_sp.at[dst_v.at[j]], add=True)
            pltpu.sync_copy(exb, den_sp.at[dst_v.at[j]], add=True)
            return carry

        lax.fori_loop(0, NB, batch, 0)

        plsc.subcore_barrier()

        for k in range(5):
            pltpu.sync_copy(out_sp.at[pl.ds(zoff + k * 128, 128)], rows)
            pltpu.sync_copy(rows, outp_h.at[c, s, pl.ds(k * 128, 128)])
            pltpu.sync_copy(den_sp.at[pl.ds(zoff + k * 128, 128)], db)
            pltpu.sync_copy(
                db, denp_h.at[pl.ds((c * 16 + s) * 640 + k * 128, 128)])

    return sc_kernel(src3, dst3, asrc, adst, h)


_B = 2048


def _tc_first(x, W, A):
    def body(x_ref, w_ref, a_ref, h_ref, al_ref):
        hblk = jnp.dot(x_ref[...], w_ref[...],
                       preferred_element_type=jnp.float32)
        h_ref[...] = hblk
        al_ref[...] = lax.dot_general(
            a_ref[...], hblk, (((0,), (1,)), ((), ())),
            preferred_element_type=jnp.float32)

    return pl.pallas_call(
        body,
        grid=(NP // _B,),
        in_specs=[
            pl.BlockSpec((_B, D), lambda i: (i, 0)),
            pl.BlockSpec((D, D), lambda i: (0, 0)),
            pl.BlockSpec((D, 2), lambda i: (0, 0)),
        ],
        out_specs=[
            pl.BlockSpec((_B, D), lambda i: (i, 0)),
            pl.BlockSpec((2, _B), lambda i: (0, i)),
        ],
        out_shape=[
            jax.ShapeDtypeStruct((NP, D), jnp.float32),
            jax.ShapeDtypeStruct((2, NP), jnp.float32),
        ],
    )(x, W, A)


def _tc_mid(p, d, b, W, A):
    def body(p_ref, d_ref, b_ref, w_ref, a_ref, h_ref, al_ref):
        den = d_ref[0] + d_ref[1] + 1e-16
        o = (p_ref[0] + p_ref[1]) / den + b_ref[...]
        hblk = jnp.dot(jnp.maximum(o, 0.0), w_ref[...],
                       preferred_element_type=jnp.float32)
        h_ref[...] = hblk
        al_ref[...] = lax.dot_general(
            a_ref[...], hblk, (((0,), (1,)), ((), ())),
            preferred_element_type=jnp.float32)

    return pl.pallas_call(
        body,
        grid=(NP // _B,),
        in_specs=[
            pl.BlockSpec((2, _B, D), lambda i: (0, i, 0)),
            pl.BlockSpec((2, _B, 1), lambda i: (0, i, 0)),
            pl.BlockSpec((1, D), lambda i: (0, 0)),
            pl.BlockSpec((D, D), lambda i: (0, 0)),
            pl.BlockSpec((D, 2), lambda i: (0, 0)),
        ],
        out_specs=[
            pl.BlockSpec((_B, D), lambda i: (i, 0)),
            pl.BlockSpec((2, _B), lambda i: (0, i)),
        ],
        out_shape=[
            jax.ShapeDtypeStruct((NP, D), jnp.float32),
            jax.ShapeDtypeStruct((2, NP), jnp.float32),
        ],
    )(p, d, b, W, A)


_BF = 2000


def _tc_final(p, d, b):
    def body(p_ref, d_ref, b_ref, o_ref):
        den = d_ref[0] + d_ref[1] + 1e-16
        o_ref[...] = (p_ref[0] + p_ref[1]) / den + b_ref[...]

    return pl.pallas_call(
        body,
        grid=(N // _BF,),
        in_specs=[
            pl.BlockSpec((2, _BF, D), lambda i: (0, i, 0)),
            pl.BlockSpec((2, _BF, 1), lambda i: (0, i, 0)),
            pl.BlockSpec((1, D), lambda i: (0, 0)),
        ],
        out_specs=pl.BlockSpec((_BF, D), lambda i: (i, 0)),
        out_shape=jax.ShapeDtypeStruct((N, D), jnp.float32),
    )(p, d, b)


def kernel(x, edge_index, W1, a1_src, a1_dst, b1, W2, a2_src, a2_dst, b2):
    src = edge_index[0].astype(jnp.int32)
    dst = edge_index[1].astype(jnp.int32)
    pidx = jnp.arange(E2 - E, dtype=jnp.int32)
    src3 = jnp.concatenate([src, (pidx * 7919) % N]).reshape(NW, NB, K)
    dst3 = jnp.concatenate([dst, N + (pidx % 8)]).reshape(NW, NB, K)
    xp = jnp.pad(x, ((0, NP - N), (0, 0)))

    A1 = jnp.stack([a1_src, a1_dst], axis=1)
    A2 = jnp.stack([a2_src, a2_dst], axis=1)

    h1, al1 = _tc_first(xp, W1, A1)
    p1, d1 = _sc_attention(src3, dst3, al1[0], al1[1], h1)

    h2, al2 = _tc_mid(p1.reshape(2, NP, D), d1.reshape(2, NP, 1),
                      b1.reshape(1, D), W2, A2)
    p2, d2 = _sc_attention(src3, dst3, al2[0], al2[1], h2)

    logits = _tc_final(p2.reshape(2, NP, D), d2.reshape(2, NP, 1),
                       b2.reshape(1, D))
    return (logits, jnp.float32(0.0))


# speedup vs baseline: 47.4477x; 1.5105x over previous
import functools

import jax
import jax.numpy as jnp
from jax import lax
from jax.experimental import pallas as pl
from jax.experimental.pallas import tpu as pltpu
from jax.experimental.pallas import tpu_sc as plsc

N = 10000
NP = 10240
D = 128
E = 320000
K = 128
NB = 80
NCB = 8
NCH = NB // NCB
NW = 32
EW2 = NB * K
E2 = NW * EW2


def _sc_attention(src3, dst3, asrc, adst, h):
    mesh = plsc.VectorSubcoreMesh(core_axis_name="c", subcore_axis_name="s")

    @functools.partial(
        pl.kernel,
        mesh=mesh,
        out_type=[
            jax.ShapeDtypeStruct((2, 16, 640, D), jnp.float32),
            jax.ShapeDtypeStruct((2 * 16 * 640,), jnp.float32),
        ],
        scratch_types=[
            pltpu.VMEM((NCB, K), jnp.int32),
            pltpu.VMEM((NCB, K), jnp.int32),
            pltpu.VMEM((NCB, K), jnp.int32),
            pltpu.VMEM((NCB, K), jnp.int32),
            pltpu.VMEM((K,), jnp.float32),
            pltpu.VMEM((K,), jnp.float32),
            pltpu.VMEM((K,), jnp.float32),
            pltpu.VMEM((K,), jnp.float32),
            pltpu.VMEM((K,), jnp.float32),
            pltpu.VMEM((K,), jnp.float32),
            pltpu.VMEM((K, D), jnp.float32),
            pltpu.VMEM((K, D), jnp.float32),
            pltpu.VMEM_SHARED((NP, D), jnp.float32),
            pltpu.VMEM_SHARED((NP,), jnp.float32),
        ] + [pltpu.SemaphoreType.DMA] * 12,
        compiler_params=pltpu.CompilerParams(needs_layout_passes=False),
    )
    def sc_kernel(src_h, dst_h, asrc_h, adst_h, h_h, outp_h, denp_h,
                  sc0, sc1, dc0, dc1, asb0, asb1, adb0, adb1, exb0, exb1,
                  rows0, rows1, out_sp, den_sp,
                  scm0, scm1, sr0, sr1, sa0, sa1, sd0, sd1,
                  ssr0, ssr1, ssd0, ssd1):
        c = lax.axis_index("c")
        s = lax.axis_index("s")
        wid = s * 2 + c
        scv = (sc0, sc1)
        dcv = (dc0, dc1)
        asb = (asb0, asb1)
        adb = (adb0, adb1)
        exb = (exb0, exb1)
        rows = (rows0, rows1)
        scm = (scm0, scm1)
        sem_r = (sr0, sr1)
        sem_a = (sa0, sa1)
        sem_d = (sd0, sd1)
        sem_sr = (ssr0, ssr1)
        sem_sd = (ssd0, ssd1)

        zvec = jnp.zeros((16,), jnp.float32)

        def zrow(r, carry):
            for cc in range(8):
                rows0[r, pl.ds(cc * 16, 16)] = zvec
            return carry

        lax.fori_loop(0, K, zrow, 0)
        for i in range(8):
            exb0[pl.ds(i * 16, 16)] = zvec

        zoff = s * 640
        for k in range(5):
            pltpu.sync_copy(rows0, out_sp.at[pl.ds(zoff + k * K, K)])
            pltpu.sync_copy(exb0, den_sp.at[pl.ds(zoff + k * 128, 128)])

        plsc.subcore_barrier()

        def fetch_chunk(ch, cb):
            pltpu.async_copy(src_h.at[wid, pl.ds(ch * NCB, NCB)],
                             scv[cb], scm[cb])
            pltpu.async_copy(dst_h.at[wid, pl.ds(ch * NCB, NCB)],
                             dcv[cb], scm[cb])

        def wait_chunk(ch, cb):
            pltpu.make_async_copy(src_h.at[wid, pl.ds(ch * NCB, NCB)],
                                  scv[cb], scm[cb]).wait()
            pltpu.make_async_copy(dst_h.at[wid, pl.ds(ch * NCB, NCB)],
                                  dcv[cb], scm[cb]).wait()

        def issue_gathers(cb, bb, rb):
            pltpu.async_copy(h_h.at[scv[cb].at[bb]], rows[rb], sem_r[rb])
            pltpu.async_copy(asrc_h.at[scv[cb].at[bb]], asb[rb], sem_a[rb])
            pltpu.async_copy(adst_h.at[dcv[cb].at[bb]], adb[rb], sem_d[rb])

        def drain_scatters(cb, bb, rb):
            pltpu.make_async_copy(
                rows[rb], out_sp.at[dcv[cb].at[bb]], sem_sr[rb]).wait()
            pltpu.make_async_copy(
                exb[rb], den_sp.at[dcv[cb].at[bb]], sem_sd[rb]).wait()

        fetch_chunk(0, 0)
        wait_chunk(0, 0)
        issue_gathers(0, 0, 0)

        def chunk_pair(ch2, carry):
            for cb in range(2):
                for bb in range(NCB):
                    rb = bb % 2
                    nrb = rb ^ 1

                    def rearm_next():
                        drain_scatters(cb, bb, nrb)
                        if bb < NCB - 1:
                            issue_gathers(cb, bb + 1, nrb)
                        else:
                            issue_gathers(cb ^ 1, 0, nrb)

                    if cb == 0 and bb == 0:
                        pl.when(ch2 > 0)(lambda: drain_scatters(cb, bb, nrb))
                        issue_gathers(cb, bb + 1, nrb)
                    elif bb == NCB - 1:
                        def cross_chunk():
                            wait_chunk(2 * ch2 + cb + 1, cb ^ 1)
                            rearm_next()
                        if cb == 0:
                            cross_chunk()
                        else:
                            pl.when(ch2 < NCH // 2 - 1)(cross_chunk)
                            pl.when(ch2 == NCH // 2 - 1)(
                                lambda: drain_scatters(cb, bb, nrb))
                    else:
                        rearm_next()

                    if bb == 2:
                        if cb == 0:
                            fetch_chunk(2 * ch2 + 1, 1)
                        else:
                            pl.when(ch2 < NCH // 2 - 1)(
                                lambda: fetch_chunk(2 * ch2 + 2, 0))

                    pltpu.make_async_copy(h_h.at[scv[cb].at[bb]],
                                          rows[rb], sem_r[rb]).wait()
                    pltpu.make_async_copy(asrc_h.at[scv[cb].at[bb]],
                                          asb[rb], sem_a[rb]).wait()
                    pltpu.make_async_copy(adst_h.at[dcv[cb].at[bb]],
                                          adb[rb], sem_d[rb]).wait()

                    for t in range(K // 16):
                        e = (asb[rb][pl.ds(t * 16, 16)]
                             + adb[rb][pl.ds(t * 16, 16)])
                        e = jnp.maximum(e, e * 0.2)
                        exb[rb][pl.ds(t * 16, 16)] = jnp.exp(e)

                    def scale(r, inner):
                        ev = plsc.load_gather(
                            exb[rb], [jnp.full((16,), r, jnp.int32)])
                        for cc in range(8):
                            rows[rb][r, pl.ds(cc * 16, 16)] = (
                                rows[rb][r, pl.ds(cc * 16, 16)] * ev)
                        return inner

                    lax.fori_loop(0, K, scale, 0, unroll=2)
                    pltpu.async_copy(rows[rb], out_sp.at[dcv[cb].at[bb]],
                                     sem_sr[rb], add=True)
                    pltpu.async_copy(exb[rb], den_sp.at[dcv[cb].at[bb]],
                                     sem_sd[rb], add=True)
            return carry

        lax.fori_loop(0, NCH // 2, chunk_pair, 0)

        drain_scatters(1, NCB - 1, 1)

        plsc.subcore_barrier()

        for k in range(5):
            pltpu.sync_copy(out_sp.at[pl.ds(zoff + k * K, K)], rows0)
            pltpu.sync_copy(rows0, outp_h.at[c, s, pl.ds(k * K, K)])
            pltpu.sync_copy(den_sp.at[pl.ds(zoff + k * 128, 128)], exb0)
            pltpu.sync_copy(
                exb0, denp_h.at[pl.ds((c * 16 + s) * 640 + k * 128, 128)])

    return sc_kernel(src3, dst3, asrc, adst, h)


_B = 2048


def _tc_first(x, W, A):
    def body(x_ref, w_ref, a_ref, h_ref, al_ref):
        hblk = jnp.dot(x_ref[...], w_ref[...],
                       preferred_element_type=jnp.float32)
        h_ref[...] = hblk
        al_ref[...] = lax.dot_general(
            a_ref[...], hblk, (((0,), (1,)), ((), ())),
            preferred_element_type=jnp.float32)

    return pl.pallas_call(
        body,
        grid=(NP // _B,),
        in_specs=[
            pl.BlockSpec((_B, D), lambda i: (i, 0)),
            pl.BlockSpec((D, D), lambda i: (0, 0)),
            pl.BlockSpec((D, 2), lambda i: (0, 0)),
        ],
        out_specs=[
            pl.BlockSpec((_B, D), lambda i: (i, 0)),
            pl.BlockSpec((2, _B), lambda i: (0, i)),
        ],
        out_shape=[
            jax.ShapeDtypeStruct((NP, D), jnp.float32),
            jax.ShapeDtypeStruct((2, NP), jnp.float32),
        ],
    )(x, W, A)


def _tc_mid(p, d, b, W, A):
    def body(p_ref, d_ref, b_ref, w_ref, a_ref, h_ref, al_ref):
        den = d_ref[0] + d_ref[1] + 1e-16
        o = (p_ref[0] + p_ref[1]) / den + b_ref[...]
        hblk = jnp.dot(jnp.maximum(o, 0.0), w_ref[...],
                       preferred_element_type=jnp.float32)
        h_ref[...] = hblk
        al_ref[...] = lax.dot_general(
            a_ref[...], hblk, (((0,), (1,)), ((), ())),
            preferred_element_type=jnp.float32)

    return pl.pallas_call(
        body,
        grid=(NP // _B,),
        in_specs=[
            pl.BlockSpec((2, _B, D), lambda i: (0, i, 0)),
            pl.BlockSpec((2, _B, 1), lambda i: (0, i, 0)),
            pl.BlockSpec((1, D), lambda i: (0, 0)),
            pl.BlockSpec((D, D), lambda i: (0, 0)),
            pl.BlockSpec((D, 2), lambda i: (0, 0)),
        ],
        out_specs=[
            pl.BlockSpec((_B, D), lambda i: (i, 0)),
            pl.BlockSpec((2, _B), lambda i: (0, i)),
        ],
        out_shape=[
            jax.ShapeDtypeStruct((NP, D), jnp.float32),
            jax.ShapeDtypeStruct((2, NP), jnp.float32),
        ],
    )(p, d, b, W, A)


_BF = 2000


def _tc_final(p, d, b):
    def body(p_ref, d_ref, b_ref, o_ref):
        den = d_ref[0] + d_ref[1] + 1e-16
        o_ref[...] = (p_ref[0] + p_ref[1]) / den + b_ref[...]

    return pl.pallas_call(
        body,
        grid=(N // _BF,),
        in_specs=[
            pl.BlockSpec((2, _BF, D), lambda i: (0, i, 0)),
            pl.BlockSpec((2, _BF, 1), lambda i: (0, i, 0)),
            pl.BlockSpec((1, D), lambda i: (0, 0)),
        ],
        out_specs=pl.BlockSpec((_BF, D), lambda i: (i, 0)),
        out_shape=jax.ShapeDtypeStruct((N, D), jnp.float32),
    )(p, d, b)


def kernel(x, edge_index, W1, a1_src, a1_dst, b1, W2, a2_src, a2_dst, b2):
    src = edge_index[0].astype(jnp.int32)
    dst = edge_index[1].astype(jnp.int32)
    pidx = jnp.arange(E2 - E, dtype=jnp.int32)
    src3 = jnp.concatenate([src, (pidx * 7919) % N]).reshape(NW, NB, K)
    dst3 = jnp.concatenate([dst, N + (pidx % 8)]).reshape(NW, NB, K)
    xp = jnp.pad(x, ((0, NP - N), (0, 0)))

    A1 = jnp.stack([a1_src, a1_dst], axis=1)
    A2 = jnp.stack([a2_src, a2_dst], axis=1)

    h1, al1 = _tc_first(xp, W1, A1)
    p1, d1 = _sc_attention(src3, dst3, al1[0], al1[1], h1)

    h2, al2 = _tc_mid(p1.reshape(2, NP, D), d1.reshape(2, NP, 1),
                      b1.reshape(1, D), W2, A2)
    p2, d2 = _sc_attention(src3, dst3, al2[0], al2[1], h2)

    logits = _tc_final(p2.reshape(2, NP, D), d2.reshape(2, NP, 1),
                       b2.reshape(1, D))
    return (logits, jnp.float32(0.0))
